# Initial kernel scaffold; baseline (speedup 1.0000x reference)
import jax
import jax.numpy as jnp
from jax.experimental import pallas as pl


def kernel(x, edge_index, Wl0, bl0, Wr0, g1, b1, Wl1, bl1, Wr1, g2, b2, Wl2, bl2, Wr2):
    raise NotImplementedError("write your pallas kernel here")


# speedup vs baseline: 7.9076x; 7.9076x over previous
import functools

import jax
import jax.numpy as jnp
from jax import lax
from jax.experimental import pallas as pl
from jax.experimental.pallas import tpu as pltpu
from jax.experimental.pallas import tpu_sc as plsc

NN = 10000
EE = 320000
DD = 128
HH = 128
PP = 64

NC = 2
NS = 16
K = 80
CH = EE // K
IPT = CH // NS
NP = 10240
RPT = NP // NS
RCH = 128
DW = 16
BN_C = 1.0 / (1.0 + 1e-5) ** 0.5


def _make_sc_agg(w, with_deg):
    mesh = plsc.VectorSubcoreMesh(core_axis_name="c", subcore_axis_name="s")

    out_type = jax.ShapeDtypeStruct((NC, NP, w), jnp.float32)
    if with_deg:
        out_type = [out_type, jax.ShapeDtypeStruct((NC, NP, DW), jnp.float32)]
    scratch = [
        pltpu.VMEM((IPT, K), jnp.int32),
        pltpu.VMEM((IPT, K), jnp.int32),
        pltpu.VMEM((K, w), jnp.float32),
        pltpu.VMEM((K, w), jnp.float32),
        pltpu.VMEM((RCH, w), jnp.float32),
        pltpu.VMEM_SHARED((NP, w), jnp.float32),
        pltpu.SemaphoreType.DMA,
        pltpu.SemaphoreType.DMA,
    ]
    if with_deg:
        scratch += [
            pltpu.VMEM((K, DW), jnp.float32),
            pltpu.VMEM((RPT, DW), jnp.float32),
            pltpu.VMEM_SHARED((NP, DW), jnp.float32),
        ]

    @functools.partial(
        pl.kernel,
        out_type=out_type,
        mesh=mesh,
        compiler_params=pltpu.CompilerParams(use_tc_tiling_on_sc=False),
        scratch_types=scratch,
    )
    def agg(z_hbm, src_hbm, dst_hbm, *refs):
        if with_deg:
            (out_hbm, outd_hbm, srcs_v, dsts_v, rows0, rows1, zbuf, acc,
             sem0, sem1, ones_v, zdeg, accd) = refs
        else:
            (out_hbm, srcs_v, dsts_v, rows0, rows1, zbuf, acc,
             sem0, sem1) = refs
        c = lax.axis_index("c")
        s = lax.axis_index("s")

        zv = jnp.zeros((16,), jnp.float32)

        def zrow(i, carry):
            for j in range(w // 16):
                zbuf[i, pl.ds(j * 16, 16)] = zv
            return carry

        lax.fori_loop(0, RCH, zrow, 0)
        for j in range(RPT // RCH):
            pltpu.sync_copy(zbuf, acc.at[pl.ds(s * RPT + j * RCH, RCH)])

        if with_deg:
            ov = jnp.ones((16,), jnp.float32)

            def drow(i, carry):
                zdeg[i, pl.ds(0, 16)] = zv
                return carry

            lax.fori_loop(0, RPT, drow, 0)
            pltpu.sync_copy(zdeg, accd.at[pl.ds(s * RPT, RPT)])
            for i in range(K):
                ones_v[i, pl.ds(0, 16)] = ov
        plsc.subcore_barrier()

        row0 = s * IPT
        pltpu.sync_copy(src_hbm.at[c, pl.ds(row0, IPT)], srcs_v)
        pltpu.sync_copy(dst_hbm.at[pl.ds(row0, IPT)], dsts_v)

        def step(i, carry):
            d0 = pltpu.async_copy(z_hbm.at[srcs_v.at[2 * i]], rows0, sem0)
            d1 = pltpu.async_copy(z_hbm.at[srcs_v.at[2 * i + 1]], rows1, sem1)
            if with_deg:
                @pl.when(c == 0)
                def _():
                    pltpu.sync_copy(ones_v, accd.at[dsts_v.at[2 * i]], add=True)

                @pl.when(c == 1)
                def _():
                    pltpu.sync_copy(ones_v, accd.at[dsts_v.at[2 * i + 1]],
                                    add=True)
            d0.wait()
            pltpu.sync_copy(rows0, acc.at[dsts_v.at[2 * i]], add=True)
            d1.wait()
            pltpu.sync_copy(rows1, acc.at[dsts_v.at[2 * i + 1]], add=True)
            return carry

        lax.fori_loop(0, IPT // 2, step, 0)
        plsc.subcore_barrier()

        for j in range(RPT // RCH):
            r0 = s * RPT + j * RCH
            pltpu.sync_copy(acc.at[pl.ds(r0, RCH)], out_hbm.at[c, pl.ds(r0, RCH)])
        if with_deg:
            pltpu.sync_copy(accd.at[pl.ds(s * RPT, RPT)],
                            outd_hbm.at[c, pl.ds(s * RPT, RPT)])

    return agg


_sc_agg_cache = {}


def _sc_agg(w, with_deg=False):
    key = (w, with_deg)
    if key not in _sc_agg_cache:
        _sc_agg_cache[key] = _make_sc_agg(w, with_deg)
    return _sc_agg_cache[key]


def _split_cols(z_ref, zflat_ref, w):
    zflat_ref[:NN] = z_ref[:, :w]
    zflat_ref[NN:] = z_ref[:, w:]


def _tc_pre_body(x_ref, wl_ref, wr_ref, bl_ref, zflat_ref, r_ref):
    x = x_ref[...]
    z = lax.dot_general(x, wl_ref[...], (((1,), (1,)), ((), ())),
                        preferred_element_type=jnp.float32)
    zflat_ref[:NN] = z[:, :HH // 2]
    zflat_ref[NN:] = z[:, HH // 2:]
    r_ref[...] = lax.dot_general(x, wr_ref[...], (((1,), (1,)), ((), ())),
                                 preferred_element_type=jnp.float32) + bl_ref[...][None, :]


def _tc_mid1_body(sp_ref, dp_ref, r_ref, g_ref, b_ref, wl_ref, wr_ref, bl_ref,
                  zflat_ref, rn_ref, invd_ref):
    sp = jnp.concatenate([sp_ref[0, :NN], sp_ref[1, :NN]], axis=1)
    deg = dp_ref[0, :NN, 0:1] + dp_ref[1, :NN, 0:1]
    invd = 1.0 / jnp.maximum(deg, 1.0)
    h = sp * invd + r_ref[...]
    h = jnp.maximum(h * (BN_C * g_ref[...])[None, :] + b_ref[...][None, :], 0.0)
    z = lax.dot_general(h, wl_ref[...], (((1,), (1,)), ((), ())),
                        preferred_element_type=jnp.float32)
    zflat_ref[:NN] = z[:, :HH // 2]
    zflat_ref[NN:] = z[:, HH // 2:]
    rn_ref[...] = lax.dot_general(h, wr_ref[...], (((1,), (1,)), ((), ())),
                                  preferred_element_type=jnp.float32) + bl_ref[...][None, :]
    invd_ref[...] = invd


def _tc_mid2_body(sp_ref, r_ref, invd_ref, g_ref, b_ref, wl_ref, wr_ref, bl_ref,
                  zflat_ref, rn_ref):
    sp = jnp.concatenate([sp_ref[0, :NN], sp_ref[1, :NN]], axis=1)
    invd = invd_ref[...]
    h = sp * invd + r_ref[...]
    h = jnp.maximum(h * (BN_C * g_ref[...])[None, :] + b_ref[...][None, :], 0.0)
    z = lax.dot_general(h, wl_ref[...], (((1,), (1,)), ((), ())),
                        preferred_element_type=jnp.float32)
    zflat_ref[:NN] = z[:, :PP // 2]
    zflat_ref[NN:] = z[:, PP // 2:]
    rn_ref[...] = lax.dot_general(h, wr_ref[...], (((1,), (1,)), ((), ())),
                                  preferred_element_type=jnp.float32) + bl_ref[...][None, :]


def _tc_fin_body(sp_ref, r_ref, invd_ref, out_ref):
    sp = jnp.concatenate([sp_ref[0, :NN], sp_ref[1, :NN]], axis=1)
    o = sp * invd_ref[...] + r_ref[...]
    m = jnp.max(o, axis=1, keepdims=True)
    e = jnp.exp(o - m)
    out_ref[...] = e / jnp.sum(e, axis=1, keepdims=True)


_f32 = jnp.float32

_tc_pre = pl.pallas_call(
    _tc_pre_body,
    out_shape=[jax.ShapeDtypeStruct((2 * NN, HH // 2), _f32),
               jax.ShapeDtypeStruct((NN, HH), _f32)],
)

_tc_mid1 = pl.pallas_call(
    _tc_mid1_body,
    out_shape=[jax.ShapeDtypeStruct((2 * NN, HH // 2), _f32),
               jax.ShapeDtypeStruct((NN, HH), _f32),
               jax.ShapeDtypeStruct((NN, 1), _f32)],
)

_tc_mid2 = pl.pallas_call(
    _tc_mid2_body,
    out_shape=[jax.ShapeDtypeStruct((2 * NN, PP // 2), _f32),
               jax.ShapeDtypeStruct((NN, PP), _f32)],
)

_tc_fin = pl.pallas_call(
    _tc_fin_body,
    out_shape=jax.ShapeDtypeStruct((NN, PP), _f32),
)


def kernel(x, edge_index, Wl0, bl0, Wr0, g1, b1, Wl1, bl1, Wr1, g2, b2, Wl2,
           bl2, Wr2):
    src = edge_index[0].reshape(CH, K)
    src2 = jnp.stack([src, src + NN])
    dst = edge_index[1].reshape(CH, K)

    zf0, r0 = _tc_pre(x, Wl0, Wr0, bl0)
    s0p, degp = _sc_agg(HH // 2, True)(zf0, src2, dst)
    zf1, r1, invd = _tc_mid1(s0p, degp, r0, g1, b1, Wl1, Wr1, bl1)
    s1p = _sc_agg(HH // 2)(zf1, src2, dst)
    zf2, r2 = _tc_mid2(s1p, r1, invd, g2, b2, Wl2, Wr2, bl2)
    s2p = _sc_agg(PP // 2)(zf2, src2, dst)
    return _tc_fin(s2p, r2, invd)


# speedup vs baseline: 13.7482x; 1.7386x over previous
import functools

import jax
import jax.numpy as jnp
from jax import lax
from jax.experimental import pallas as pl
from jax.experimental.pallas import tpu as pltpu
from jax.experimental.pallas import tpu_sc as plsc

NN = 10000
EE = 320000
DD = 128
HH = 128
PP = 64

NC = 2
NS = 16
K = 125
CH = EE // K
IPT = CH // NS
NB = 4
NP = 10240
RPT = NP // NS
RCH = 128
DW = 16
BN_C = 1.0 / (1.0 + 1e-5) ** 0.5


def _make_sc_agg(w, with_deg):
    mesh = plsc.VectorSubcoreMesh(core_axis_name="c", subcore_axis_name="s")

    out_type = jax.ShapeDtypeStruct((NC, NP, w), jnp.float32)
    if with_deg:
        out_type = [out_type, jax.ShapeDtypeStruct((NC, NP, DW), jnp.float32)]
    scratch = [
        pltpu.VMEM((IPT, K), jnp.int32),
        pltpu.VMEM((IPT, K), jnp.int32),
        [pltpu.VMEM((K, w), jnp.float32)] * NB,
        pltpu.VMEM_SHARED((NP, w), jnp.float32),
        pltpu.SemaphoreType.DMA,
        [pltpu.SemaphoreType.DMA] * NB,
    ]
    if with_deg:
        scratch += [
            pltpu.VMEM((K, DW), jnp.float32),
            pltpu.VMEM((RCH, DW), jnp.float32),
            pltpu.VMEM_SHARED((NP, DW), jnp.float32),
        ]

    @functools.partial(
        pl.kernel,
        out_type=out_type,
        mesh=mesh,
        compiler_params=pltpu.CompilerParams(use_tc_tiling_on_sc=False),
        scratch_types=scratch,
    )
    def agg(z_hbm, src_hbm, dst_hbm, *refs):
        if with_deg:
            (out_hbm, outd_hbm, srcs_v, dsts_v, rows, acc,
             isem, gsems, ones_v, zdeg, accd) = refs
        else:
            (out_hbm, srcs_v, dsts_v, rows, acc, isem, gsems) = refs
        c = lax.axis_index("c")
        s = lax.axis_index("s")

        row0 = s * IPT
        di = pltpu.async_copy(src_hbm.at[c, pl.ds(row0, IPT)], srcs_v, isem)
        dj = pltpu.async_copy(dst_hbm.at[pl.ds(row0, IPT)], dsts_v, isem)

        zv = jnp.zeros((16,), jnp.float32)

        def zrow(i, carry):
            for j in range(w // 16):
                rows[0][i, pl.ds(j * 16, 16)] = zv
            return carry

        lax.fori_loop(0, K, zrow, 0)
        offs = list(range(0, RPT - K + 1, K))
        if offs[-1] != RPT - K:
            offs.append(RPT - K)
        for o in offs:
            pltpu.sync_copy(rows[0], acc.at[pl.ds(s * RPT + o, K)])

        if with_deg:
            ov = jnp.ones((16,), jnp.float32)

            def drow(i, carry):
                zdeg[i, pl.ds(0, 16)] = zv
                return carry

            lax.fori_loop(0, RCH, drow, 0)
            for j in range(RPT // RCH):
                pltpu.sync_copy(zdeg, accd.at[pl.ds(s * RPT + j * RCH, RCH)])
            for i in range(K):
                ones_v[i, pl.ds(0, 16)] = ov

        di.wait()
        dj.wait()
        for b in range(NB):
            pltpu.async_copy(z_hbm.at[srcs_v.at[b]], rows[b], gsems[b])
        plsc.subcore_barrier()

        def wait_gather(b):
            pltpu.make_async_copy(z_hbm.at[pl.ds(0, K)], rows[b],
                                  gsems[b]).wait()

        def chunk_deg(b, idx):
            if with_deg:
                @pl.when(c == (b % 2))
                def _():
                    pltpu.sync_copy(ones_v, accd.at[dsts_v.at[idx]], add=True)

        def step(i, carry):
            for b in range(NB):
                idx = NB * i + b
                wait_gather(b)
                chunk_deg(b, idx)
                pltpu.sync_copy(rows[b], acc.at[dsts_v.at[idx]], add=True)
                pltpu.async_copy(z_hbm.at[srcs_v.at[NB * (i + 1) + b]],
                                 rows[b], gsems[b])
            return carry

        lax.fori_loop(0, IPT // NB - 1, step, 0)
        for b in range(NB):
            idx = IPT - NB + b
            wait_gather(b)
            chunk_deg(b, idx)
            pltpu.sync_copy(rows[b], acc.at[dsts_v.at[idx]], add=True)
        plsc.subcore_barrier()

        for j in range(RPT // RCH):
            r0 = s * RPT + j * RCH
            pltpu.sync_copy(acc.at[pl.ds(r0, RCH)], out_hbm.at[c, pl.ds(r0, RCH)])
        if with_deg:
            pltpu.sync_copy(accd.at[pl.ds(s * RPT, RPT)],
                            outd_hbm.at[c, pl.ds(s * RPT, RPT)])

    return agg


_sc_agg_cache = {}


def _sc_agg(w, with_deg=False):
    key = (w, with_deg)
    if key not in _sc_agg_cache:
        _sc_agg_cache[key] = _make_sc_agg(w, with_deg)
    return _sc_agg_cache[key]


def _split_cols(z_ref, zflat_ref, w):
    zflat_ref[:NN] = z_ref[:, :w]
    zflat_ref[NN:] = z_ref[:, w:]


def _tc_pre_body(x_ref, wl_ref, wr_ref, bl_ref, zflat_ref, r_ref):
    x = x_ref[...]
    z = lax.dot_general(x, wl_ref[...], (((1,), (1,)), ((), ())),
                        preferred_element_type=jnp.float32)
    zflat_ref[:NN] = z[:, :HH // 2]
    zflat_ref[NN:] = z[:, HH // 2:]
    r_ref[...] = lax.dot_general(x, wr_ref[...], (((1,), (1,)), ((), ())),
                                 preferred_element_type=jnp.float32) + bl_ref[...][None, :]


def _tc_mid1_body(sp_ref, dp_ref, r_ref, g_ref, b_ref, wl_ref, wr_ref, bl_ref,
                  zflat_ref, rn_ref, invd_ref):
    sp = jnp.concatenate([sp_ref[0, :NN], sp_ref[1, :NN]], axis=1)
    deg = dp_ref[0, :NN, 0:1] + dp_ref[1, :NN, 0:1]
    invd = 1.0 / jnp.maximum(deg, 1.0)
    h = sp * invd + r_ref[...]
    h = jnp.maximum(h * (BN_C * g_ref[...])[None, :] + b_ref[...][None, :], 0.0)
    z = lax.dot_general(h, wl_ref[...], (((1,), (1,)), ((), ())),
                        preferred_element_type=jnp.float32)
    zflat_ref[:NN] = z[:, :HH // 2]
    zflat_ref[NN:] = z[:, HH // 2:]
    rn_ref[...] = lax.dot_general(h, wr_ref[...], (((1,), (1,)), ((), ())),
                                  preferred_element_type=jnp.float32) + bl_ref[...][None, :]
    invd_ref[...] = invd


def _tc_mid2_body(sp_ref, r_ref, invd_ref, g_ref, b_ref, wl_ref, wr_ref, bl_ref,
                  zflat_ref, rn_ref):
    sp = jnp.concatenate([sp_ref[0, :NN], sp_ref[1, :NN]], axis=1)
    invd = invd_ref[...]
    h = sp * invd + r_ref[...]
    h = jnp.maximum(h * (BN_C * g_ref[...])[None, :] + b_ref[...][None, :], 0.0)
    z = lax.dot_general(h, wl_ref[...], (((1,), (1,)), ((), ())),
                        preferred_element_type=jnp.float32)
    zflat_ref[:NN] = z[:, :PP // 2]
    zflat_ref[NN:] = z[:, PP // 2:]
    rn_ref[...] = lax.dot_general(h, wr_ref[...], (((1,), (1,)), ((), ())),
                                  preferred_element_type=jnp.float32) + bl_ref[...][None, :]


def _tc_fin_body(sp_ref, r_ref, invd_ref, out_ref):
    sp = jnp.concatenate([sp_ref[0, :NN], sp_ref[1, :NN]], axis=1)
    o = sp * invd_ref[...] + r_ref[...]
    m = jnp.max(o, axis=1, keepdims=True)
    e = jnp.exp(o - m)
    out_ref[...] = e / jnp.sum(e, axis=1, keepdims=True)


_f32 = jnp.float32

_tc_pre = pl.pallas_call(
    _tc_pre_body,
    out_shape=[jax.ShapeDtypeStruct((2 * NN, HH // 2), _f32),
               jax.ShapeDtypeStruct((NN, HH), _f32)],
)

_tc_mid1 = pl.pallas_call(
    _tc_mid1_body,
    out_shape=[jax.ShapeDtypeStruct((2 * NN, HH // 2), _f32),
               jax.ShapeDtypeStruct((NN, HH), _f32),
               jax.ShapeDtypeStruct((NN, 1), _f32)],
)

_tc_mid2 = pl.pallas_call(
    _tc_mid2_body,
    out_shape=[jax.ShapeDtypeStruct((2 * NN, PP // 2), _f32),
               jax.ShapeDtypeStruct((NN, PP), _f32)],
)

_tc_fin = pl.pallas_call(
    _tc_fin_body,
    out_shape=jax.ShapeDtypeStruct((NN, PP), _f32),
)


def kernel(x, edge_index, Wl0, bl0, Wr0, g1, b1, Wl1, bl1, Wr1, g2, b2, Wl2,
           bl2, Wr2):
    src = edge_index[0].reshape(CH, K)
    src2 = jnp.stack([src, src + NN])
    dst = edge_index[1].reshape(CH, K)

    zf0, r0 = _tc_pre(x, Wl0, Wr0, bl0)
    s0p, degp = _sc_agg(HH // 2, True)(zf0, src2, dst)
    zf1, r1, invd = _tc_mid1(s0p, degp, r0, g1, b1, Wl1, Wr1, bl1)
    s1p = _sc_agg(HH // 2)(zf1, src2, dst)
    zf2, r2 = _tc_mid2(s1p, r1, invd, g2, b2, Wl2, Wr2, bl2)
    s2p = _sc_agg(PP // 2)(zf2, src2, dst)
    return _tc_fin(s2p, r2, invd)
